# baseline (speedup 1.0000x reference)
import jax
import jax.numpy as jnp
import numpy as np
from jax import lax
from jax.experimental import pallas as pl
from jax.experimental.pallas import tpu as pltpu
from jax.experimental.pallas import tpu_sc as plsc

B, N, D, K = 8, 8192, 512, 2048
NC, NS = 2, 16
RPC = B // NC
WPR = NS // RPC
IPW = K // WPR
CHUNK = 64
RADIX_BITS = 8
RADIX = 1 << RADIX_BITS
PASSES = 32 // RADIX_BITS
SIGN = np.uint32(0x80000000)
NOSIGN = np.uint32(0x7FFFFFFF)


def _sc_body(w_hbm, f_hbm, c_hbm, fout_hbm, cout_hbm, wout_hbm,
             wrow, kk0, id0, kk1, id1, hist, off, wtop, crow, cbuf,
             idx_v, gidx_v, wv, fbuf, idx_sh, w_sh, sem):
  c = lax.axis_index("c")
  s = lax.axis_index("s")
  iota = lax.iota(jnp.int32, 16)
  ones = jnp.ones((16,), jnp.int32)

  @pl.when(s < RPC)
  def _sort_phase():
    r = c * RPC + s
    pltpu.sync_copy(w_hbm.at[pl.ds(r * N, N)], wrow)

    def build(i, _):
      sl = pl.ds(i * 16, 16)
      w16 = wrow[sl]
      u = plsc.bitcast(w16, jnp.uint32)
      neg = plsc.bitcast(w16, jnp.int32) < 0
      kk = jnp.where(neg, u, (~u) & NOSIGN)
      kk0[sl] = plsc.bitcast(kk, jnp.int32)
      id0[sl] = iota + i * 16
      return 0
    with jax.named_scope("p1_build"):
      lax.fori_loop(0, N // 16, build, 0, unroll=4)

    for p in range(PASSES):
      kin, iin, kout, iout = ((kk0, id0, kk1, id1) if p % 2 == 0
                              else (kk1, id1, kk0, id0))
      shift = p * RADIX_BITS
      mask = jnp.uint32(RADIX - 1)

      def zero(i, _):
        hist[pl.ds(i * 16, 16)] = jnp.zeros((16,), jnp.int32)
        return 0
      lax.fori_loop(0, RADIX // 16, zero, 0, unroll=4)

      def histo(i, _, kin=kin, shift=shift, mask=mask):
        kk = plsc.bitcast(kin[pl.ds(i * 16, 16)], jnp.uint32)
        d = ((kk >> shift) & mask).astype(jnp.int32)
        plsc.addupdate_scatter(hist, [d], ones)
        return 0
      with jax.named_scope("p1_hist"):
        lax.fori_loop(0, N // 16, histo, 0, unroll=4)

      def scan(i, carry):
        sl = pl.ds(i * 16, 16)
        h = hist[sl]
        cs = plsc.cumsum(h)
        off[sl] = cs - h + carry
        return carry + jnp.sum(h)
      lax.fori_loop(0, RADIX // 16, scan, jnp.int32(0))

      def perm(i, _, kin=kin, iin=iin, kout=kout, iout=iout,
               shift=shift, mask=mask):
        sl = pl.ds(i * 16, 16)
        kki = kin[sl]
        idi = iin[sl]
        kk = plsc.bitcast(kki, jnp.uint32)
        d = ((kk >> shift) & mask).astype(jnp.int32)
        basev = plsc.load_gather(off, [d])
        cnt, _ = plsc.scan_count(d)
        pos = basev + cnt - 1
        plsc.store_scatter(kout, [pos], kki)
        plsc.store_scatter(iout, [pos], idi)
        plsc.addupdate_scatter(off, [d], ones)
        return 0
      with jax.named_scope("p1_perm"):
        lax.fori_loop(0, N // 16, perm, 0, unroll=2)

    kfin, ifin = (kk0, id0) if PASSES % 2 == 0 else (kk1, id1)

    def unkey(i, _):
      sl = pl.ds(i * 16, 16)
      kk = plsc.bitcast(kfin[sl], jnp.uint32)
      negk = kk >= SIGN
      u = jnp.where(negk, kk, (~kk) & NOSIGN)
      wtop[sl] = plsc.bitcast(u, jnp.float32)
      return 0
    lax.fori_loop(0, K // 16, unkey, 0, unroll=4)
    pltpu.sync_copy(wtop, wout_hbm.at[pl.ds(r * K, K)])

    for k3 in range(3):
      pltpu.sync_copy(c_hbm.at[pl.ds((k3 * B + r) * N, N)],
                      crow.at[pl.ds(k3 * N, N)])

    def cgather(i, _):
      j16 = iota + i * 16
      idx16 = ifin[pl.ds(i * 16, 16)]
      for k3 in range(3):
        vals = plsc.load_gather(crow, [idx16 + (k3 * N)])
        plsc.store_scatter(cbuf, [j16 * 3 + k3], vals)
      return 0
    with jax.named_scope("p1_cgather"):
      lax.fori_loop(0, K // 16, cgather, 0, unroll=2)
    pltpu.sync_copy(cbuf, cout_hbm.at[pl.ds(r * K * 3, K * 3)])

    pltpu.sync_copy(ifin.at[pl.ds(0, K)], idx_sh.at[pl.ds(s * K, K)])
    pltpu.sync_copy(wtop, w_sh.at[pl.ds(s * K, K)])

  plsc.subcore_barrier()

  lr = s // WPR
  q = s - lr * WPR
  r = c * RPC + lr
  out_base = r * K + q * IPW
  pltpu.sync_copy(idx_sh.at[pl.ds(lr * K + q * IPW, IPW)], idx_v)
  pltpu.sync_copy(w_sh.at[pl.ds(lr * K + q * IPW, IPW)], wv)

  def glob(i, _):
    sl = pl.ds(i * 16, 16)
    gidx_v[sl] = idx_v[sl] + r * N
    return 0
  lax.fori_loop(0, IPW // 16, glob, 0, unroll=4)

  for t in range(IPW // CHUNK):
    pltpu.async_copy(
        f_hbm.at[gidx_v.at[pl.ds(t * CHUNK, CHUNK)]], fbuf, sem).wait()

    def scale(i, _, t=t):
      j = i >> 5
      m = i - (j << 5)
      rows = jnp.full((16,), 0, jnp.int32) + j
      cols = iota + m * 16
      wj = plsc.load_gather(wv, [jnp.full((16,), t * CHUNK, jnp.int32) + j])
      v = plsc.load_gather(fbuf, [rows, cols])
      plsc.store_scatter(fbuf, [rows, cols], v * wj)
      return 0
    with jax.named_scope("p2_scale"):
      lax.fori_loop(0, CHUNK * (D // 16), scale, 0, unroll=8)

    pltpu.sync_copy(fbuf, fout_hbm.at[pl.ds(out_base + t * CHUNK, CHUNK)])


def _get_kernel():
  mesh = plsc.VectorSubcoreMesh(core_axis_name="c", subcore_axis_name="s",
                                num_cores=NC, num_subcores=NS)
  return pl.kernel(
      _sc_body,
      out_type=(jax.ShapeDtypeStruct((B * K, D), jnp.float32),
                jax.ShapeDtypeStruct((B * K * 3,), jnp.float32),
                jax.ShapeDtypeStruct((B * K,), jnp.float32)),
      mesh=mesh,
      compiler_params=pltpu.CompilerParams(needs_layout_passes=False),
      scratch_types=[
          pltpu.VMEM((N,), jnp.float32),
          pltpu.VMEM((N,), jnp.int32),
          pltpu.VMEM((N,), jnp.int32),
          pltpu.VMEM((N,), jnp.int32),
          pltpu.VMEM((N,), jnp.int32),
          pltpu.VMEM((RADIX,), jnp.int32),
          pltpu.VMEM((RADIX,), jnp.int32),
          pltpu.VMEM((K,), jnp.float32),
          pltpu.VMEM((3 * N,), jnp.float32),
          pltpu.VMEM((K * 3,), jnp.float32),
          pltpu.VMEM((IPW,), jnp.int32),
          pltpu.VMEM((IPW,), jnp.int32),
          pltpu.VMEM((IPW,), jnp.float32),
          pltpu.VMEM((CHUNK, D), jnp.float32),
          pltpu.VMEM_SHARED((RPC * K,), jnp.int32),
          pltpu.VMEM_SHARED((RPC * K,), jnp.float32),
          pltpu.SemaphoreType.DMA,
      ],
  )


def kernel(node_weights, node_feats, coords):
  w1 = node_weights.reshape(B * N)
  f2 = node_feats.reshape(B * N, D)
  ct = jnp.transpose(coords, (2, 0, 1)).reshape(3 * B * N)
  fout, cout, wout = _get_kernel()(w1, f2, ct)
  return (fout.reshape(B, K, D), cout.reshape(B, K, 3),
          wout.reshape(B, K))


# speedup vs baseline: 1.7056x; 1.7056x over previous
import jax
import jax.numpy as jnp
import numpy as np
from jax import lax
from jax.experimental import pallas as pl
from jax.experimental.pallas import tpu as pltpu
from jax.experimental.pallas import tpu_sc as plsc

B, N, D, K = 8, 8192, 512, 2048
NC, NS = 2, 16
RPC = B // NC
WPR = NS // RPC
IPW = K // WPR
CHUNK = 32
RADIX_BITS = 8
RADIX = 1 << RADIX_BITS
PASSES = 32 // RADIX_BITS
SIGN = np.uint32(0x80000000)
NOSIGN = np.uint32(0x7FFFFFFF)


def _sc_body(w_hbm, f_hbm, c_hbm, fout_hbm, cout_hbm, wout_hbm,
             wrow, kk0, id0, kk1, id1, hist, off, wtop, crow, cbuf,
             idx_v, gidx_v, wv, fbuf, fbuf2, idx_sh, w_sh,
             sem, gsem2, wsem, wsem2):
  c = lax.axis_index("c")
  s = lax.axis_index("s")
  iota = lax.iota(jnp.int32, 16)
  ones = jnp.ones((16,), jnp.int32)

  @pl.when(s < RPC)
  def _sort_phase():
    r = c * RPC + s
    pltpu.sync_copy(w_hbm.at[pl.ds(r * N, N)], wrow)

    def build(i, _):
      sl = pl.ds(i * 16, 16)
      w16 = wrow[sl]
      u = plsc.bitcast(w16, jnp.uint32)
      neg = plsc.bitcast(w16, jnp.int32) < 0
      kk = jnp.where(neg, u, (~u) & NOSIGN)
      kk0[sl] = plsc.bitcast(kk, jnp.int32)
      id0[sl] = iota + i * 16
      return 0
    with jax.named_scope("p1_build"):
      lax.fori_loop(0, N // 16, build, 0, unroll=4)

    for p in range(PASSES):
      kin, iin, kout, iout = ((kk0, id0, kk1, id1) if p % 2 == 0
                              else (kk1, id1, kk0, id0))
      shift = p * RADIX_BITS
      mask = jnp.uint32(RADIX - 1)

      def zero(i, _):
        hist[pl.ds(i * 16, 16)] = jnp.zeros((16,), jnp.int32)
        return 0
      lax.fori_loop(0, RADIX // 16, zero, 0, unroll=4)

      def histo(i, _, kin=kin, shift=shift, mask=mask):
        kk = plsc.bitcast(kin[pl.ds(i * 16, 16)], jnp.uint32)
        d = ((kk >> shift) & mask).astype(jnp.int32)
        plsc.addupdate_scatter(hist, [d], ones)
        return 0
      with jax.named_scope("p1_hist"):
        lax.fori_loop(0, N // 16, histo, 0, unroll=4)

      def scan(i, carry):
        sl = pl.ds(i * 16, 16)
        h = hist[sl]
        cs = plsc.cumsum(h)
        off[sl] = cs - h + carry
        return carry + jnp.sum(h)
      lax.fori_loop(0, RADIX // 16, scan, jnp.int32(0))

      def perm(i, _, kin=kin, iin=iin, kout=kout, iout=iout,
               shift=shift, mask=mask):
        sl = pl.ds(i * 16, 16)
        kki = kin[sl]
        idi = iin[sl]
        kk = plsc.bitcast(kki, jnp.uint32)
        d = ((kk >> shift) & mask).astype(jnp.int32)
        basev = plsc.load_gather(off, [d])
        cnt, _ = plsc.scan_count(d)
        pos = basev + cnt - 1
        plsc.store_scatter(kout, [pos], kki)
        plsc.store_scatter(iout, [pos], idi)
        plsc.addupdate_scatter(off, [d], ones)
        return 0
      with jax.named_scope("p1_perm"):
        lax.fori_loop(0, N // 16, perm, 0, unroll=2)

    kfin, ifin = (kk0, id0) if PASSES % 2 == 0 else (kk1, id1)

    def unkey(i, _):
      sl = pl.ds(i * 16, 16)
      kk = plsc.bitcast(kfin[sl], jnp.uint32)
      negk = kk >= SIGN
      u = jnp.where(negk, kk, (~kk) & NOSIGN)
      wtop[sl] = plsc.bitcast(u, jnp.float32)
      return 0
    lax.fori_loop(0, K // 16, unkey, 0, unroll=4)
    pltpu.sync_copy(wtop, wout_hbm.at[pl.ds(r * K, K)])

    for k3 in range(3):
      pltpu.sync_copy(c_hbm.at[pl.ds((k3 * B + r) * N, N)],
                      crow.at[pl.ds(k3 * N, N)])

    def cgather(i, _):
      j16 = iota + i * 16
      idx16 = ifin[pl.ds(i * 16, 16)]
      for k3 in range(3):
        vals = plsc.load_gather(crow, [idx16 + (k3 * N)])
        plsc.store_scatter(cbuf, [j16 * 3 + k3], vals)
      return 0
    with jax.named_scope("p1_cgather"):
      lax.fori_loop(0, K // 16, cgather, 0, unroll=2)
    pltpu.sync_copy(cbuf, cout_hbm.at[pl.ds(r * K * 3, K * 3)])

    pltpu.sync_copy(ifin.at[pl.ds(0, K)], idx_sh.at[pl.ds(s * K, K)])
    pltpu.sync_copy(wtop, w_sh.at[pl.ds(s * K, K)])

  plsc.subcore_barrier()

  lr = s // WPR
  q = s - lr * WPR
  r = c * RPC + lr
  out_base = r * K + q * IPW
  pltpu.sync_copy(idx_sh.at[pl.ds(lr * K + q * IPW, IPW)], idx_v)
  pltpu.sync_copy(w_sh.at[pl.ds(lr * K + q * IPW, IPW)], wv)

  def glob(i, _):
    sl = pl.ds(i * 16, 16)
    gidx_v[sl] = idx_v[sl] + r * N
    return 0
  lax.fori_loop(0, IPW // 16, glob, 0, unroll=4)

  nch = IPW // CHUNK
  bufs = (fbuf, fbuf2)
  gsems = (sem, gsem2)
  wsems = (wsem, wsem2)

  def _scale(buf, t):
    def rowscale(j, _):
      wj = plsc.load_gather(wv, [jnp.full((16,), t * CHUNK, jnp.int32) + j])
      for m in range(D // 16):
        sl = pl.ds(m * 16, 16)
        buf[j, sl] = buf[j, sl] * wj
      return 0
    with jax.named_scope("p2_scale"):
      lax.fori_loop(0, CHUNK, rowscale, 0)

  gcp = [None] * nch
  wcp = [None] * nch
  gcp[0] = pltpu.async_copy(
      f_hbm.at[gidx_v.at[pl.ds(0, CHUNK)]], bufs[0], gsems[0])
  for t in range(nch):
    b = t % 2
    if t + 1 < nch:
      if t >= 1:
        wcp[t - 1].wait()
      gcp[t + 1] = pltpu.async_copy(
          f_hbm.at[gidx_v.at[pl.ds((t + 1) * CHUNK, CHUNK)]],
          bufs[1 - b], gsems[1 - b])
    gcp[t].wait()
    _scale(bufs[b], t)
    wcp[t] = pltpu.async_copy(
        bufs[b], fout_hbm.at[pl.ds(out_base + t * CHUNK, CHUNK)], wsems[b])
  wcp[nch - 2].wait()
  wcp[nch - 1].wait()


def _get_kernel():
  mesh = plsc.VectorSubcoreMesh(core_axis_name="c", subcore_axis_name="s",
                                num_cores=NC, num_subcores=NS)
  return pl.kernel(
      _sc_body,
      out_type=(jax.ShapeDtypeStruct((B * K, D), jnp.float32),
                jax.ShapeDtypeStruct((B * K * 3,), jnp.float32),
                jax.ShapeDtypeStruct((B * K,), jnp.float32)),
      mesh=mesh,
      compiler_params=pltpu.CompilerParams(needs_layout_passes=False),
      scratch_types=[
          pltpu.VMEM((N,), jnp.float32),
          pltpu.VMEM((N,), jnp.int32),
          pltpu.VMEM((N,), jnp.int32),
          pltpu.VMEM((N,), jnp.int32),
          pltpu.VMEM((N,), jnp.int32),
          pltpu.VMEM((RADIX,), jnp.int32),
          pltpu.VMEM((RADIX,), jnp.int32),
          pltpu.VMEM((K,), jnp.float32),
          pltpu.VMEM((3 * N,), jnp.float32),
          pltpu.VMEM((K * 3,), jnp.float32),
          pltpu.VMEM((IPW,), jnp.int32),
          pltpu.VMEM((IPW,), jnp.int32),
          pltpu.VMEM((IPW,), jnp.float32),
          pltpu.VMEM((CHUNK, D), jnp.float32),
          pltpu.VMEM((CHUNK, D), jnp.float32),
          pltpu.VMEM_SHARED((RPC * K,), jnp.int32),
          pltpu.VMEM_SHARED((RPC * K,), jnp.float32),
          pltpu.SemaphoreType.DMA,
          pltpu.SemaphoreType.DMA,
          pltpu.SemaphoreType.DMA,
          pltpu.SemaphoreType.DMA,
      ],
  )


def kernel(node_weights, node_feats, coords):
  w1 = node_weights.reshape(B * N)
  f2 = node_feats.reshape(B * N, D)
  ct = jnp.transpose(coords, (2, 0, 1)).reshape(3 * B * N)
  fout, cout, wout = _get_kernel()(w1, f2, ct)
  return (fout.reshape(B, K, D), cout.reshape(B, K, 3),
          wout.reshape(B, K))


# speedup vs baseline: 1.8529x; 1.0864x over previous
import jax
import jax.numpy as jnp
import numpy as np
from jax import lax
from jax.experimental import pallas as pl
from jax.experimental.pallas import tpu as pltpu
from jax.experimental.pallas import tpu_sc as plsc

B, N, D, K = 8, 8192, 512, 2048
NC, NS = 2, 16
RPC = B // NC
WPR = NS // RPC
IPW = K // WPR
CHUNK = 32
RADIX_BITS = 8
RADIX = 1 << RADIX_BITS
PASSES = 32 // RADIX_BITS
SIGN = np.uint32(0x80000000)
NOSIGN = np.uint32(0x7FFFFFFF)


def _sc_body(w_hbm, f_hbm, c_hbm, fout_hbm, cout_hbm, wout_hbm,
             wrow, kk0, id0, kk1, id1, hist, off, wtop, crow, cbuf,
             idx_v, gidx_v, wv, fbuf, fbuf2, idx_sh, w_sh,
             sem, gsem2, wsem, wsem2):
  c = lax.axis_index("c")
  s = lax.axis_index("s")
  iota = lax.iota(jnp.int32, 16)
  ones = jnp.ones((16,), jnp.int32)

  @pl.when(s < RPC)
  def _sort_phase():
    r = c * RPC + s
    pltpu.sync_copy(w_hbm.at[pl.ds(r * N, N)], wrow)

    def build(i, _):
      sl = pl.ds(i * 16, 16)
      w16 = wrow[sl]
      u = plsc.bitcast(w16, jnp.uint32)
      neg = plsc.bitcast(w16, jnp.int32) < 0
      kk = jnp.where(neg, u, (~u) & NOSIGN)
      kk0[sl] = plsc.bitcast(kk, jnp.int32)
      id0[sl] = iota + i * 16
      return 0
    with jax.named_scope("p1_build"):
      lax.fori_loop(0, N // 16, build, 0, unroll=4)

    mask8 = jnp.uint32(RADIX - 1)

    def _zero_hist():
      def zero(i, _):
        hist[pl.ds(i * 16, 16)] = jnp.zeros((16,), jnp.int32)
        return 0
      lax.fori_loop(0, RADIX // 16, zero, 0, unroll=4)

    def _cum_and_find(remaining):
      def scan(i, carry):
        sl = pl.ds(i * 16, 16)
        cs = plsc.cumsum(hist[sl])
        off[sl] = cs + carry
        return carry + jnp.max(cs)
      lax.fori_loop(0, RADIX // 16, scan, jnp.int32(0))

      def find(i, dd):
        return dd + jnp.sum((off[pl.ds(i * 16, 16)] < remaining)
                            .astype(jnp.int32))
      dd = lax.fori_loop(0, RADIX // 16, find, jnp.int32(0))
      before16 = plsc.load_gather(off, [jnp.maximum(dd - 1, 0)
                                        + jnp.zeros((16,), jnp.int32)])
      cnt_before = jnp.where(dd > 0, jnp.max(before16), jnp.int32(0))
      return dd, cnt_before

    with jax.named_scope("p1_thresh"):
      remaining = jnp.int32(K)
      _zero_hist()

      def hist3(i, _):
        kk = plsc.bitcast(kk0[pl.ds(i * 16, 16)], jnp.uint32)
        d = (kk >> 24).astype(jnp.int32)
        plsc.addupdate_scatter(hist, [d], ones)
        return 0
      lax.fori_loop(0, N // 16, hist3, 0, unroll=4)
      dd, cnt_before = _cum_and_find(remaining)
      remaining = remaining - cnt_before
      prefix = dd.astype(jnp.uint32) << 24

      def extract(i, nc):
        sl = pl.ds(i * 16, 16)
        kki = kk0[sl]
        kk = plsc.bitcast(kki, jnp.uint32)
        sel = (kk >> 24).astype(jnp.int32) == dd
        scnt = plsc.cumsum(sel.astype(jnp.int32))
        pos = nc + scnt - 1
        plsc.store_scatter(kk1, [pos], kki, mask=sel)
        return nc + jnp.max(scnt)
      nc = lax.fori_loop(0, N // 16, extract, jnp.int32(0))

      for l in (2, 1, 0):
        sh = 8 * l
        nv = (nc + 15) // 16
        _zero_hist()

        def histl(i, _, sh=sh, nc=nc):
          sl = pl.ds(i * 16, 16)
          ck = plsc.bitcast(kk1[sl], jnp.uint32)
          d = ((ck >> sh) & mask8).astype(jnp.int32)
          valid = (iota + i * 16) < nc
          plsc.addupdate_scatter(hist, [d], ones, mask=valid)
          return 0
        lax.fori_loop(0, nv, histl, 0)
        dd, cnt_before = _cum_and_find(remaining)
        remaining = remaining - cnt_before
        prefix = prefix | (dd.astype(jnp.uint32) << sh)

        def compactl(i, nc2, sh=sh, nc=nc, dd=dd):
          sl = pl.ds(i * 16, 16)
          cki = kk1[sl]
          ck = plsc.bitcast(cki, jnp.uint32)
          d = ((ck >> sh) & mask8).astype(jnp.int32)
          sel = (d == dd) & ((iota + i * 16) < nc)
          scnt = plsc.cumsum(sel.astype(jnp.int32))
          pos = nc2 + scnt - 1
          plsc.store_scatter(kk1, [pos], cki, mask=sel)
          return nc2 + jnp.max(scnt)
        nc = lax.fori_loop(0, nv, compactl, jnp.int32(0))

      T = prefix
      needed = remaining

    with jax.named_scope("p1_compact"):
      def fcompact(i, carry):
        out_count, eq_count = carry
        sl = pl.ds(i * 16, 16)
        kki = kk0[sl]
        idv = id0[sl]
        kk = plsc.bitcast(kki, jnp.uint32)
        lt = kk < T
        eq = kk == T
        eqc = plsc.cumsum(eq.astype(jnp.int32)) + eq_count
        sel = lt | (eq & (eqc <= needed))
        scnt = plsc.cumsum(sel.astype(jnp.int32))
        pos = out_count + scnt - 1
        plsc.store_scatter(kk1, [pos], kki, mask=sel)
        plsc.store_scatter(id1, [pos], idv, mask=sel)
        return out_count + jnp.max(scnt), jnp.max(eqc)
      lax.fori_loop(0, N // 16, fcompact, (jnp.int32(0), jnp.int32(0)),
                    unroll=2)

    for p in range(PASSES):
      kin, iin, kout, iout = ((kk1, id1, kk0, id0) if p % 2 == 0
                              else (kk0, id0, kk1, id1))
      shift = p * RADIX_BITS
      _zero_hist()

      def histo(i, _, kin=kin, shift=shift):
        kk = plsc.bitcast(kin[pl.ds(i * 16, 16)], jnp.uint32)
        d = ((kk >> shift) & mask8).astype(jnp.int32)
        plsc.addupdate_scatter(hist, [d], ones)
        return 0
      with jax.named_scope("p1_hist"):
        lax.fori_loop(0, K // 16, histo, 0, unroll=4)

      def scan(i, carry):
        sl = pl.ds(i * 16, 16)
        h = hist[sl]
        cs = plsc.cumsum(h)
        off[sl] = cs - h + carry
        return carry + jnp.max(cs)
      lax.fori_loop(0, RADIX // 16, scan, jnp.int32(0))

      def perm(i, _, kin=kin, iin=iin, kout=kout, iout=iout, shift=shift):
        sl = pl.ds(i * 16, 16)
        kki = kin[sl]
        idi = iin[sl]
        kk = plsc.bitcast(kki, jnp.uint32)
        d = ((kk >> shift) & mask8).astype(jnp.int32)
        basev = plsc.load_gather(off, [d])
        cnt, _ = plsc.scan_count(d)
        pos = basev + cnt - 1
        plsc.store_scatter(kout, [pos], kki)
        plsc.store_scatter(iout, [pos], idi)
        plsc.addupdate_scatter(off, [d], ones)
        return 0
      with jax.named_scope("p1_perm"):
        lax.fori_loop(0, K // 16, perm, 0, unroll=2)

    kfin, ifin = (kk1, id1) if PASSES % 2 == 0 else (kk0, id0)

    def unkey(i, _):
      sl = pl.ds(i * 16, 16)
      kk = plsc.bitcast(kfin[sl], jnp.uint32)
      negk = kk >= SIGN
      u = jnp.where(negk, kk, (~kk) & NOSIGN)
      wtop[sl] = plsc.bitcast(u, jnp.float32)
      return 0
    lax.fori_loop(0, K // 16, unkey, 0, unroll=4)
    pltpu.sync_copy(wtop, wout_hbm.at[pl.ds(r * K, K)])

    for k3 in range(3):
      pltpu.sync_copy(c_hbm.at[pl.ds((k3 * B + r) * N, N)],
                      crow.at[pl.ds(k3 * N, N)])

    def cgather(i, _):
      j16 = iota + i * 16
      idx16 = ifin[pl.ds(i * 16, 16)]
      for k3 in range(3):
        vals = plsc.load_gather(crow, [idx16 + (k3 * N)])
        plsc.store_scatter(cbuf, [j16 * 3 + k3], vals)
      return 0
    with jax.named_scope("p1_cgather"):
      lax.fori_loop(0, K // 16, cgather, 0, unroll=2)
    pltpu.sync_copy(cbuf, cout_hbm.at[pl.ds(r * K * 3, K * 3)])

    pltpu.sync_copy(ifin.at[pl.ds(0, K)], idx_sh.at[pl.ds(s * K, K)])
    pltpu.sync_copy(wtop, w_sh.at[pl.ds(s * K, K)])

  plsc.subcore_barrier()

  lr = s // WPR
  q = s - lr * WPR
  r = c * RPC + lr
  out_base = r * K + q * IPW
  pltpu.sync_copy(idx_sh.at[pl.ds(lr * K + q * IPW, IPW)], idx_v)
  pltpu.sync_copy(w_sh.at[pl.ds(lr * K + q * IPW, IPW)], wv)

  def glob(i, _):
    sl = pl.ds(i * 16, 16)
    gidx_v[sl] = idx_v[sl] + r * N
    return 0
  lax.fori_loop(0, IPW // 16, glob, 0, unroll=4)

  nch = IPW // CHUNK
  bufs = (fbuf, fbuf2)
  gsems = (sem, gsem2)
  wsems = (wsem, wsem2)

  def _scale(buf, t):
    def rowscale(j, _):
      wj = plsc.load_gather(wv, [jnp.full((16,), t * CHUNK, jnp.int32) + j])
      for m in range(D // 16):
        sl = pl.ds(m * 16, 16)
        buf[j, sl] = buf[j, sl] * wj
      return 0
    with jax.named_scope("p2_scale"):
      lax.fori_loop(0, CHUNK, rowscale, 0)

  gcp = [None] * nch
  wcp = [None] * nch
  gcp[0] = pltpu.async_copy(
      f_hbm.at[gidx_v.at[pl.ds(0, CHUNK)]], bufs[0], gsems[0])
  for t in range(nch):
    b = t % 2
    if t + 1 < nch:
      if t >= 1:
        wcp[t - 1].wait()
      gcp[t + 1] = pltpu.async_copy(
          f_hbm.at[gidx_v.at[pl.ds((t + 1) * CHUNK, CHUNK)]],
          bufs[1 - b], gsems[1 - b])
    gcp[t].wait()
    _scale(bufs[b], t)
    wcp[t] = pltpu.async_copy(
        bufs[b], fout_hbm.at[pl.ds(out_base + t * CHUNK, CHUNK)], wsems[b])
  wcp[nch - 2].wait()
  wcp[nch - 1].wait()


def _get_kernel():
  mesh = plsc.VectorSubcoreMesh(core_axis_name="c", subcore_axis_name="s",
                                num_cores=NC, num_subcores=NS)
  return pl.kernel(
      _sc_body,
      out_type=(jax.ShapeDtypeStruct((B * K, D), jnp.float32),
                jax.ShapeDtypeStruct((B * K * 3,), jnp.float32),
                jax.ShapeDtypeStruct((B * K,), jnp.float32)),
      mesh=mesh,
      compiler_params=pltpu.CompilerParams(needs_layout_passes=False),
      scratch_types=[
          pltpu.VMEM((N,), jnp.float32),
          pltpu.VMEM((N,), jnp.int32),
          pltpu.VMEM((N,), jnp.int32),
          pltpu.VMEM((N,), jnp.int32),
          pltpu.VMEM((N,), jnp.int32),
          pltpu.VMEM((RADIX,), jnp.int32),
          pltpu.VMEM((RADIX,), jnp.int32),
          pltpu.VMEM((K,), jnp.float32),
          pltpu.VMEM((3 * N,), jnp.float32),
          pltpu.VMEM((K * 3,), jnp.float32),
          pltpu.VMEM((IPW,), jnp.int32),
          pltpu.VMEM((IPW,), jnp.int32),
          pltpu.VMEM((IPW,), jnp.float32),
          pltpu.VMEM((CHUNK, D), jnp.float32),
          pltpu.VMEM((CHUNK, D), jnp.float32),
          pltpu.VMEM_SHARED((RPC * K,), jnp.int32),
          pltpu.VMEM_SHARED((RPC * K,), jnp.float32),
          pltpu.SemaphoreType.DMA,
          pltpu.SemaphoreType.DMA,
          pltpu.SemaphoreType.DMA,
          pltpu.SemaphoreType.DMA,
      ],
  )


def kernel(node_weights, node_feats, coords):
  w1 = node_weights.reshape(B * N)
  f2 = node_feats.reshape(B * N, D)
  ct = jnp.transpose(coords, (2, 0, 1)).reshape(3 * B * N)
  fout, cout, wout = _get_kernel()(w1, f2, ct)
  return (fout.reshape(B, K, D), cout.reshape(B, K, 3),
          wout.reshape(B, K))


# speedup vs baseline: 2.2158x; 1.1959x over previous
import jax
import jax.numpy as jnp
import numpy as np
from jax import lax
from jax.experimental import pallas as pl
from jax.experimental.pallas import tpu as pltpu
from jax.experimental.pallas import tpu_sc as plsc

B, N, D, K = 8, 8192, 512, 2048
NC, NS = 2, 16
RPC = B // NC
WPR = NS // RPC
Q = N // WPR
IPW = K // WPR
CHUNK = 32
RADIX_BITS = 8
RADIX = 1 << RADIX_BITS
XB = 2 * RADIX
SBITS = 11
SRADIX = 1 << SBITS
SXB = SRADIX + 16
SMASK = np.uint32(SRADIX - 1)
PASSES = 32 // RADIX_BITS
SIGN = np.uint32(0x80000000)
NOSIGN = np.uint32(0x7FFFFFFF)
MASK8 = np.uint32(RADIX - 1)


def _sc_body(w_hbm, f_hbm, c_hbm, fout_hbm, cout_hbm, wout_hbm,
             kk0, id0, kk1, id1, hist, off, hist2, off2, wtop, hmerge, cntv, nqv,
             crow, cbuf, idx_v, gidx_v, wv, fbuf, fbuf2,
             h_sh, cnt_sh, ck_sh, ci_sh, idx_sh, w_sh,
             sem, gsem2, wsem, wsem2):
  c = lax.axis_index("c")
  s = lax.axis_index("s")
  iota = lax.iota(jnp.int32, 16)
  ones = jnp.ones((16,), jnp.int32)
  lr = s // WPR
  q = s - lr * WPR
  r = c * RPC + lr
  qbase = q * Q

  def _zero_hist():
    def zero(i, _):
      hist[pl.ds(i * 16, 16)] = jnp.zeros((16,), jnp.int32)
      return 0
    lax.fori_loop(0, XB // 16, zero, 0, unroll=4)

  def _publish_hist_and_merge():
    pltpu.sync_copy(hist.at[pl.ds(0, RADIX)], h_sh.at[pl.ds(s * RADIX, RADIX)])
    plsc.subcore_barrier()
    pltpu.sync_copy(h_sh.at[pl.ds(lr * WPR * RADIX, WPR * RADIX)], hmerge)
    def merge(i, _):
      sl = pl.ds(i * 16, 16)
      m = (hmerge[sl] + hmerge[pl.ds(RADIX + i * 16, 16)]
           + hmerge[pl.ds(2 * RADIX + i * 16, 16)]
           + hmerge[pl.ds(3 * RADIX + i * 16, 16)])
      hist[sl] = m
      return 0
    lax.fori_loop(0, RADIX // 16, merge, 0, unroll=4)
    plsc.subcore_barrier()

  def _cum_and_find(remaining):
    def scan(i, carry):
      sl = pl.ds(i * 16, 16)
      cs = plsc.cumsum(hist[sl])
      off[sl] = cs + carry
      return carry + jnp.max(cs)
    lax.fori_loop(0, RADIX // 16, scan, jnp.int32(0))

    def find(i, dd):
      return dd + jnp.sum((off[pl.ds(i * 16, 16)] < remaining)
                          .astype(jnp.int32))
    dd = lax.fori_loop(0, RADIX // 16, find, jnp.int32(0))
    before16 = plsc.load_gather(off, [jnp.maximum(dd - 1, 0)
                                      + jnp.zeros((16,), jnp.int32)])
    cnt_before = jnp.where(dd > 0, jnp.max(before16), jnp.int32(0))
    return dd, cnt_before

  pltpu.sync_copy(w_hbm.at[pl.ds(r * N + qbase, Q)], wtop)

  def build(i, _):
    sl = pl.ds(i * 16, 16)
    w16 = wtop[sl]
    u = plsc.bitcast(w16, jnp.uint32)
    neg = plsc.bitcast(w16, jnp.int32) < 0
    kk = jnp.where(neg, u, (~u) & NOSIGN)
    kk0[sl] = plsc.bitcast(kk, jnp.int32)
    id0[sl] = iota + (i * 16 + qbase)
    return 0
  with jax.named_scope("p1_build"):
    lax.fori_loop(0, Q // 16, build, 0, unroll=4)

  with jax.named_scope("p1_thresh"):
    remaining = jnp.int32(K)
    _zero_hist()

    def hist3(i, _):
      kk = plsc.bitcast(kk0[pl.ds(i * 16, 16)], jnp.uint32)
      d = (kk >> 24).astype(jnp.int32)
      plsc.addupdate_scatter(hist, [d], ones)
      return 0
    lax.fori_loop(0, Q // 16, hist3, 0, unroll=4)
    _publish_hist_and_merge()
    dd, cnt_before = _cum_and_find(remaining)
    remaining = remaining - cnt_before
    prefix = dd.astype(jnp.uint32) << 24

    def extract(i, nc):
      sl = pl.ds(i * 16, 16)
      kki = kk0[sl]
      kk = plsc.bitcast(kki, jnp.uint32)
      sel = (kk >> 24).astype(jnp.int32) == dd
      scnt = plsc.cumsum(sel.astype(jnp.int32))
      pos = nc + scnt - 1
      plsc.store_scatter(kk1, [pos], kki, mask=sel)
      return nc + jnp.max(scnt)
    nc = lax.fori_loop(0, Q // 16, extract, jnp.int32(0))

    for l in (2, 1, 0):
      sh = 8 * l
      nv = (nc + 15) // 16
      _zero_hist()

      def histl(i, _, sh=sh, nc=nc):
        sl = pl.ds(i * 16, 16)
        ck = plsc.bitcast(kk1[sl], jnp.uint32)
        d = ((ck >> sh) & MASK8).astype(jnp.int32)
        valid = (iota + i * 16) < nc
        d2 = jnp.where(valid, d, RADIX + iota)
        plsc.addupdate_scatter(hist, [d2], ones)
        return 0
      lax.fori_loop(0, nv, histl, 0)
      _publish_hist_and_merge()
      dd, cnt_before = _cum_and_find(remaining)
      remaining = remaining - cnt_before
      prefix = prefix | (dd.astype(jnp.uint32) << sh)

      def compactl(i, nc2, sh=sh, nc=nc, dd=dd):
        sl = pl.ds(i * 16, 16)
        cki = kk1[sl]
        ck = plsc.bitcast(cki, jnp.uint32)
        d = ((ck >> sh) & MASK8).astype(jnp.int32)
        sel = (d == dd) & ((iota + i * 16) < nc)
        scnt = plsc.cumsum(sel.astype(jnp.int32))
        pos = nc2 + scnt - 1
        plsc.store_scatter(kk1, [pos], cki, mask=sel)
        return nc2 + jnp.max(scnt)
      nc = lax.fori_loop(0, nv, compactl, jnp.int32(0))

    T = prefix

  with jax.named_scope("p1_compact"):
    def fcompact(i, nq):
      sl = pl.ds(i * 16, 16)
      kki = kk0[sl]
      idv = id0[sl]
      kk = plsc.bitcast(kki, jnp.uint32)
      sel = kk <= T
      scnt = plsc.cumsum(sel.astype(jnp.int32))
      pos = nq + scnt - 1
      plsc.store_scatter(kk1, [pos], kki, mask=sel)
      plsc.store_scatter(id1, [pos], idv, mask=sel)
      return nq + jnp.max(scnt)
    nq = lax.fori_loop(0, Q // 16, fcompact, jnp.int32(0), unroll=2)

    pltpu.sync_copy(kk1.at[pl.ds(0, Q)], ck_sh.at[pl.ds(s * Q, Q)])
    pltpu.sync_copy(id1.at[pl.ds(0, Q)], ci_sh.at[pl.ds(s * Q, Q)])
    nqv[...] = jnp.zeros((16,), jnp.int32) + nq
    pltpu.sync_copy(nqv, cnt_sh.at[pl.ds(s * 16, 16)])
  plsc.subcore_barrier()

  @pl.when(q == 0)
  def _sort_phase():
    pltpu.sync_copy(cnt_sh.at[pl.ds(lr * WPR * 16, WPR * 16)], cntv)
    nqs = [jnp.max(cntv[pl.ds(j * 16, 16)]) for j in range(WPR)]
    pltpu.sync_copy(ck_sh.at[pl.ds(lr * WPR * Q, WPR * Q)], kk0)
    pltpu.sync_copy(ci_sh.at[pl.ds(lr * WPR * Q, WPR * Q)], id0)
    ntot = nqs[0] + nqs[1] + nqs[2] + nqs[3]

    def _nq_of(jj):
      return jnp.max(plsc.load_gather(cntv, [jj * 16 + jnp.zeros((16,),
                                                                 jnp.int32)]))

    def _pass(shift, segmented, kin, iin, kout, iout):
      def zero2(i, _):
        hist2[pl.ds(i * 16, 16)] = jnp.zeros((16,), jnp.int32)
        return 0
      lax.fori_loop(0, SXB // 16, zero2, 0, unroll=4)

      def _hist_seg(base, nn):
        def histo(i, _):
          sl = pl.ds(base + i * 16, 16)
          kk = plsc.bitcast(kin[sl], jnp.uint32)
          d = ((kk >> shift) & SMASK).astype(jnp.int32)
          valid = (iota + i * 16) < nn
          d2 = jnp.where(valid, d, SRADIX + iota)
          plsc.addupdate_scatter(hist2, [d2], ones)
          return 0
        lax.fori_loop(0, (nn + 15) // 16, histo, 0)

      def _perm_seg(base, nn):
        def perm(i, _):
          sl = pl.ds(base + i * 16, 16)
          kki = kin[sl]
          idi = iin[sl]
          kk = plsc.bitcast(kki, jnp.uint32)
          d = ((kk >> shift) & SMASK).astype(jnp.int32)
          valid = (iota + i * 16) < nn
          d2 = jnp.where(valid, d, SRADIX + iota)
          basev = plsc.load_gather(off2, [d2])
          cnt, _ = plsc.scan_count(d2)
          pos = basev + cnt - 1
          plsc.store_scatter(kout, [pos], kki, mask=valid)
          plsc.store_scatter(iout, [pos], idi, mask=valid)
          plsc.addupdate_scatter(off2, [d2], ones)
          return 0
        lax.fori_loop(0, (nn + 15) // 16, perm, 0)

      with jax.named_scope("p1_hist"):
        if segmented:
          for jj in range(WPR):
            _hist_seg(jj * Q, nqs[jj])
        else:
          _hist_seg(0, ntot)

      def scan2(i, carry):
        sl = pl.ds(i * 16, 16)
        h = hist2[sl]
        cs = plsc.cumsum(h)
        off2[sl] = cs - h + carry
        return carry + jnp.max(cs)
      lax.fori_loop(0, SXB // 16, scan2, jnp.int32(0))

      with jax.named_scope("p1_perm"):
        if segmented:
          for jj in range(WPR):
            _perm_seg(jj * Q, nqs[jj])
        else:
          _perm_seg(0, ntot)

    _pass(0, True, kk0, id0, kk1, id1)

    def dense_pass(t, _):
      _pass(SBITS * (t + 1), False, kk1, id1, kk0, id0)
      pltpu.sync_copy(kk0, ck_sh.at[pl.ds(lr * WPR * Q, WPR * Q)])
      pltpu.sync_copy(id0, ci_sh.at[pl.ds(lr * WPR * Q, WPR * Q)])
      pltpu.sync_copy(ck_sh.at[pl.ds(lr * WPR * Q, WPR * Q)], kk1)
      pltpu.sync_copy(ci_sh.at[pl.ds(lr * WPR * Q, WPR * Q)], id1)
      return 0
    lax.fori_loop(0, 2, dense_pass, 0)

    def unkey(i, _):
      sl = pl.ds(i * 16, 16)
      kk = plsc.bitcast(kk1[sl], jnp.uint32)
      negk = kk >= SIGN
      u = jnp.where(negk, kk, (~kk) & NOSIGN)
      wtop[sl] = plsc.bitcast(u, jnp.float32)
      return 0
    lax.fori_loop(0, K // 16, unkey, 0, unroll=4)
    pltpu.sync_copy(wtop.at[pl.ds(0, K)], wout_hbm.at[pl.ds(r * K, K)])
    pltpu.sync_copy(id1.at[pl.ds(0, K)], idx_sh.at[pl.ds(lr * K, K)])
    pltpu.sync_copy(wtop.at[pl.ds(0, K)], w_sh.at[pl.ds(lr * K, K)])

  plsc.subcore_barrier()

  out_base = r * K + q * IPW
  pltpu.sync_copy(idx_sh.at[pl.ds(lr * K + q * IPW, IPW)], idx_v)
  pltpu.sync_copy(w_sh.at[pl.ds(lr * K + q * IPW, IPW)], wv)

  def glob(i, _):
    sl = pl.ds(i * 16, 16)
    gidx_v[sl] = idx_v[sl] + r * N
    return 0
  lax.fori_loop(0, IPW // 16, glob, 0, unroll=4)

  nch = IPW // CHUNK
  npairs = nch // 2
  FB = CHUNK * D * 4

  def _gissue(t, buf, gsem):
    return pltpu.async_copy(f_hbm.at[gidx_v.at[pl.ds(t * CHUNK, CHUNK)]],
                            buf, gsem)

  def _scale(buf, t):
    def rowscale(j, _):
      wj = plsc.load_gather(wv, [t * CHUNK + j + jnp.zeros((16,), jnp.int32)])
      def colscale(m, _):
        sl = pl.ds(m * 16, 16)
        buf[j, sl] = buf[j, sl] * wj
        return 0
      lax.fori_loop(0, D // 16, colscale, 0, unroll=8)
      return 0
    with jax.named_scope("p2_scale"):
      lax.fori_loop(0, CHUNK, rowscale, 0)

  def _wait(ref_from, ref_to, sm):
    pltpu.make_async_copy(ref_from, ref_to, sm).wait()

  _gissue(0, fbuf, sem)
  _gissue(1, fbuf2, gsem2)

  for k3 in range(3):
    pltpu.sync_copy(c_hbm.at[pl.ds((k3 * B + r) * N, N)],
                    crow.at[pl.ds(k3 * N, N)])

  def cgather(i, _):
    j16 = iota + i * 16
    idx16 = idx_v[pl.ds(i * 16, 16)]
    for k3 in range(3):
      vals = plsc.load_gather(crow, [idx16 + (k3 * N)])
      plsc.store_scatter(cbuf, [j16 * 3 + k3], vals)
    return 0
  with jax.named_scope("p2_coords"):
    lax.fori_loop(0, IPW // 16, cgather, 0, unroll=2)
  pltpu.sync_copy(cbuf, cout_hbm.at[pl.ds(out_base * 3, IPW * 3)])

  def pairbody(t, _):
    o0 = out_base + (2 * t) * CHUNK
    o1 = out_base + (2 * t + 1) * CHUNK
    _wait(f_hbm.at[pl.ds(0, CHUNK)], fbuf, sem)
    _scale(fbuf, 2 * t)
    pltpu.async_copy(fbuf, fout_hbm.at[pl.ds(o0, CHUNK)], wsem)
    _wait(f_hbm.at[pl.ds(0, CHUNK)], fbuf2, gsem2)
    _scale(fbuf2, 2 * t + 1)
    pltpu.async_copy(fbuf2, fout_hbm.at[pl.ds(o1, CHUNK)], wsem2)

    @pl.when(t + 1 < npairs)
    def _():
      _wait(fbuf, fout_hbm.at[pl.ds(0, CHUNK)], wsem)
      _gissue(2 * t + 2, fbuf, sem)
      _wait(fbuf2, fout_hbm.at[pl.ds(0, CHUNK)], wsem2)
      _gissue(2 * t + 3, fbuf2, gsem2)
    return 0

  lax.fori_loop(0, npairs, pairbody, 0)
  _wait(fbuf, fout_hbm.at[pl.ds(0, CHUNK)], wsem)
  _wait(fbuf2, fout_hbm.at[pl.ds(0, CHUNK)], wsem2)


def _get_kernel():
  mesh = plsc.VectorSubcoreMesh(core_axis_name="c", subcore_axis_name="s",
                                num_cores=NC, num_subcores=NS)
  return pl.kernel(
      _sc_body,
      out_type=(jax.ShapeDtypeStruct((B * K, D), jnp.float32),
                jax.ShapeDtypeStruct((B * K * 3,), jnp.float32),
                jax.ShapeDtypeStruct((B * K,), jnp.float32)),
      mesh=mesh,
      compiler_params=pltpu.CompilerParams(needs_layout_passes=False),
      scratch_types=[
          pltpu.VMEM((N,), jnp.int32),
          pltpu.VMEM((N,), jnp.int32),
          pltpu.VMEM((N,), jnp.int32),
          pltpu.VMEM((N,), jnp.int32),
          pltpu.VMEM((XB,), jnp.int32),
          pltpu.VMEM((XB,), jnp.int32),
          pltpu.VMEM((SXB,), jnp.int32),
          pltpu.VMEM((SXB,), jnp.int32),
          pltpu.VMEM((Q,), jnp.float32),
          pltpu.VMEM((WPR * RADIX,), jnp.int32),
          pltpu.VMEM((WPR * 16,), jnp.int32),
          pltpu.VMEM((16,), jnp.int32),
          pltpu.VMEM((3 * N,), jnp.float32),
          pltpu.VMEM((IPW * 3,), jnp.float32),
          pltpu.VMEM((IPW,), jnp.int32),
          pltpu.VMEM((IPW,), jnp.int32),
          pltpu.VMEM((IPW,), jnp.float32),
          pltpu.VMEM((CHUNK, D), jnp.float32),
          pltpu.VMEM((CHUNK, D), jnp.float32),
          pltpu.VMEM_SHARED((NS * RADIX,), jnp.int32),
          pltpu.VMEM_SHARED((NS * 16,), jnp.int32),
          pltpu.VMEM_SHARED((NS * Q,), jnp.int32),
          pltpu.VMEM_SHARED((NS * Q,), jnp.int32),
          pltpu.VMEM_SHARED((RPC * K,), jnp.int32),
          pltpu.VMEM_SHARED((RPC * K,), jnp.float32),
          pltpu.SemaphoreType.DMA,
          pltpu.SemaphoreType.DMA,
          pltpu.SemaphoreType.DMA,
          pltpu.SemaphoreType.DMA,
      ],
  )


def kernel(node_weights, node_feats, coords):
  w1 = node_weights.reshape(B * N)
  f2 = node_feats.reshape(B * N, D)
  ct = jnp.transpose(coords, (2, 0, 1)).reshape(3 * B * N)
  fout, cout, wout = _get_kernel()(w1, f2, ct)
  return (fout.reshape(B, K, D), cout.reshape(B, K, 3),
          wout.reshape(B, K))


# speedup vs baseline: 2.4012x; 1.0837x over previous
import jax
import jax.numpy as jnp
import numpy as np
from jax import lax
from jax.experimental import pallas as pl
from jax.experimental.pallas import tpu as pltpu
from jax.experimental.pallas import tpu_sc as plsc

B, N, D, K = 8, 8192, 512, 2048
NC, NS = 2, 16
RPC = B // NC
WPR = NS // RPC
Q = N // WPR
IPW = K // WPR
CHUNK = 32
RADIX_BITS = 8
RADIX = 1 << RADIX_BITS
XB = 2 * RADIX
SBITS = 11
SRADIX = 1 << SBITS
SXB = SRADIX + 16
SMASK = np.uint32(SRADIX - 1)
PASSES = 32 // RADIX_BITS
SIGN = np.uint32(0x80000000)
NOSIGN = np.uint32(0x7FFFFFFF)
MASK8 = np.uint32(RADIX - 1)


def _sc_body(w_hbm, f_hbm, c_hbm, fout_hbm, cout_hbm, wout_hbm,
             kk0, id0, kk1, id1, hist, off, hist2, off2, wtop, hmerge, cntv, nqv,
             crow, cbuf, idx_v, gidx_v, wv, fbuf, fbuf2,
             h_sh, cnt_sh, ck_sh, ci_sh, idx_sh, w_sh,
             sem, gsem2, wsem, wsem2, csem):
  c = lax.axis_index("c")
  s = lax.axis_index("s")
  iota = lax.iota(jnp.int32, 16)
  ones = jnp.ones((16,), jnp.int32)
  lr = s // WPR
  q = s - lr * WPR
  r = c * RPC + lr
  qbase = q * Q

  def _zero_hist():
    def zero(i, _):
      hist[pl.ds(i * 16, 16)] = jnp.zeros((16,), jnp.int32)
      return 0
    lax.fori_loop(0, XB // 16, zero, 0, unroll=4)

  def _publish_hist_and_merge():
    pltpu.sync_copy(hist.at[pl.ds(0, RADIX)], h_sh.at[pl.ds(s * RADIX, RADIX)])
    plsc.subcore_barrier()
    pltpu.sync_copy(h_sh.at[pl.ds(lr * WPR * RADIX, WPR * RADIX)], hmerge)
    def merge(i, _):
      sl = pl.ds(i * 16, 16)
      m = (hmerge[sl] + hmerge[pl.ds(RADIX + i * 16, 16)]
           + hmerge[pl.ds(2 * RADIX + i * 16, 16)]
           + hmerge[pl.ds(3 * RADIX + i * 16, 16)])
      hist[sl] = m
      return 0
    lax.fori_loop(0, RADIX // 16, merge, 0, unroll=4)
    plsc.subcore_barrier()

  def _cum_and_find(remaining):
    def scan(i, carry):
      sl = pl.ds(i * 16, 16)
      cs = plsc.cumsum(hist[sl])
      off[sl] = cs + carry
      return carry + jnp.max(cs)
    lax.fori_loop(0, RADIX // 16, scan, jnp.int32(0))

    def find(i, dd):
      return dd + jnp.sum((off[pl.ds(i * 16, 16)] < remaining)
                          .astype(jnp.int32))
    dd = lax.fori_loop(0, RADIX // 16, find, jnp.int32(0))
    before16 = plsc.load_gather(off, [jnp.maximum(dd - 1, 0)
                                      + jnp.zeros((16,), jnp.int32)])
    cnt_before = jnp.where(dd > 0, jnp.max(before16), jnp.int32(0))
    return dd, cnt_before

  pltpu.sync_copy(w_hbm.at[pl.ds(r * N + qbase, Q)], wtop)
  ccps = [pltpu.async_copy(c_hbm.at[pl.ds((k3 * B + r) * N, N)],
                           crow.at[pl.ds(k3 * N, N)], csem)
          for k3 in range(3)]

  def build(i, _):
    sl = pl.ds(i * 16, 16)
    w16 = wtop[sl]
    u = plsc.bitcast(w16, jnp.uint32)
    neg = plsc.bitcast(w16, jnp.int32) < 0
    kk = jnp.where(neg, u, (~u) & NOSIGN)
    kk0[sl] = plsc.bitcast(kk, jnp.int32)
    id0[sl] = iota + (i * 16 + qbase)
    return 0
  with jax.named_scope("p1_build"):
    lax.fori_loop(0, Q // 16, build, 0, unroll=4)

  with jax.named_scope("p1_thresh"):
    remaining = jnp.int32(K)
    _zero_hist()

    def hist3(i, _):
      kk = plsc.bitcast(kk0[pl.ds(i * 16, 16)], jnp.uint32)
      d = (kk >> 24).astype(jnp.int32)
      plsc.addupdate_scatter(hist, [d], ones)
      return 0
    lax.fori_loop(0, Q // 16, hist3, 0, unroll=4)
    _publish_hist_and_merge()
    dd, cnt_before = _cum_and_find(remaining)
    remaining = remaining - cnt_before
    prefix = dd.astype(jnp.uint32) << 24

    def extract(i, nc):
      sl = pl.ds(i * 16, 16)
      kki = kk0[sl]
      kk = plsc.bitcast(kki, jnp.uint32)
      sel = (kk >> 24).astype(jnp.int32) == dd
      scnt = plsc.cumsum(sel.astype(jnp.int32))
      pos = nc + scnt - 1
      plsc.store_scatter(kk1, [pos], kki, mask=sel)
      return nc + jnp.max(scnt)
    nc = lax.fori_loop(0, Q // 16, extract, jnp.int32(0))

    for l in (2,):
      sh = 8 * l
      nv = (nc + 15) // 16
      _zero_hist()

      def histl(i, _, sh=sh, nc=nc):
        sl = pl.ds(i * 16, 16)
        ck = plsc.bitcast(kk1[sl], jnp.uint32)
        d = ((ck >> sh) & MASK8).astype(jnp.int32)
        valid = (iota + i * 16) < nc
        d2 = jnp.where(valid, d, RADIX + iota)
        plsc.addupdate_scatter(hist, [d2], ones)
        return 0
      lax.fori_loop(0, nv, histl, 0)
      _publish_hist_and_merge()
      dd, cnt_before = _cum_and_find(remaining)
      remaining = remaining - cnt_before
      prefix = prefix | (dd.astype(jnp.uint32) << sh)

      def compactl(i, nc2, sh=sh, nc=nc, dd=dd):
        sl = pl.ds(i * 16, 16)
        cki = kk1[sl]
        ck = plsc.bitcast(cki, jnp.uint32)
        d = ((ck >> sh) & MASK8).astype(jnp.int32)
        sel = (d == dd) & ((iota + i * 16) < nc)
        scnt = plsc.cumsum(sel.astype(jnp.int32))
        pos = nc2 + scnt - 1
        plsc.store_scatter(kk1, [pos], cki, mask=sel)
        return nc2 + jnp.max(scnt)
      nc = lax.fori_loop(0, nv, compactl, jnp.int32(0))

    T = prefix | jnp.uint32(0xFFFF)

  with jax.named_scope("p1_compact"):
    def fcompact(i, nq):
      sl = pl.ds(i * 16, 16)
      kki = kk0[sl]
      idv = id0[sl]
      kk = plsc.bitcast(kki, jnp.uint32)
      sel = kk <= T
      scnt = plsc.cumsum(sel.astype(jnp.int32))
      pos = nq + scnt - 1
      plsc.store_scatter(kk1, [pos], kki, mask=sel)
      plsc.store_scatter(id1, [pos], idv, mask=sel)
      return nq + jnp.max(scnt)
    nq = lax.fori_loop(0, Q // 16, fcompact, jnp.int32(0), unroll=2)

    pltpu.sync_copy(kk1.at[pl.ds(0, Q)], ck_sh.at[pl.ds(s * Q, Q)])
    pltpu.sync_copy(id1.at[pl.ds(0, Q)], ci_sh.at[pl.ds(s * Q, Q)])
    nqv[...] = jnp.zeros((16,), jnp.int32) + nq
    pltpu.sync_copy(nqv, cnt_sh.at[pl.ds(s * 16, 16)])
  plsc.subcore_barrier()

  @pl.when(q == 0)
  def _sort_phase():
    pltpu.sync_copy(cnt_sh.at[pl.ds(lr * WPR * 16, WPR * 16)], cntv)
    nqs = [jnp.max(cntv[pl.ds(j * 16, 16)]) for j in range(WPR)]
    pltpu.sync_copy(ck_sh.at[pl.ds(lr * WPR * Q, WPR * Q)], kk0)
    pltpu.sync_copy(ci_sh.at[pl.ds(lr * WPR * Q, WPR * Q)], id0)
    ntot = nqs[0] + nqs[1] + nqs[2] + nqs[3]

    def _nq_of(jj):
      return jnp.max(plsc.load_gather(cntv, [jj * 16 + jnp.zeros((16,),
                                                                 jnp.int32)]))

    def _pass(shift, segmented, kin, iin, kout, iout):
      def zero2(i, _):
        hist2[pl.ds(i * 16, 16)] = jnp.zeros((16,), jnp.int32)
        return 0
      lax.fori_loop(0, SXB // 16, zero2, 0, unroll=4)

      def _hist_seg(base, nn):
        def histo(i, _):
          sl = pl.ds(base + i * 16, 16)
          kk = plsc.bitcast(kin[sl], jnp.uint32)
          d = ((kk >> shift) & SMASK).astype(jnp.int32)
          valid = (iota + i * 16) < nn
          d2 = jnp.where(valid, d, SRADIX + iota)
          plsc.addupdate_scatter(hist2, [d2], ones)
          return 0
        lax.fori_loop(0, (nn + 15) // 16, histo, 0)

      def _perm_seg(base, nn):
        def perm(i, _):
          sl = pl.ds(base + i * 16, 16)
          kki = kin[sl]
          idi = iin[sl]
          kk = plsc.bitcast(kki, jnp.uint32)
          d = ((kk >> shift) & SMASK).astype(jnp.int32)
          valid = (iota + i * 16) < nn
          d2 = jnp.where(valid, d, SRADIX + iota)
          basev = plsc.load_gather(off2, [d2])
          cnt, _ = plsc.scan_count(d2)
          pos = basev + cnt - 1
          plsc.store_scatter(kout, [pos], kki, mask=valid)
          plsc.store_scatter(iout, [pos], idi, mask=valid)
          plsc.addupdate_scatter(off2, [d2], ones)
          return 0
        lax.fori_loop(0, (nn + 15) // 16, perm, 0)

      with jax.named_scope("p1_hist"):
        if segmented:
          for jj in range(WPR):
            _hist_seg(jj * Q, nqs[jj])
        else:
          _hist_seg(0, ntot)

      def scan2(i, carry):
        sl = pl.ds(i * 16, 16)
        h = hist2[sl]
        cs = plsc.cumsum(h)
        off2[sl] = cs - h + carry
        return carry + jnp.max(cs)
      lax.fori_loop(0, SXB // 16, scan2, jnp.int32(0))

      with jax.named_scope("p1_perm"):
        if segmented:
          for jj in range(WPR):
            _perm_seg(jj * Q, nqs[jj])
        else:
          _perm_seg(0, ntot)

    _pass(0, True, kk0, id0, kk1, id1)

    def dense_pass(t, _):
      _pass(SBITS * (t + 1), False, kk1, id1, kk0, id0)

      @pl.when(t == 0)
      def _():
        pltpu.sync_copy(kk0, ck_sh.at[pl.ds(lr * WPR * Q, WPR * Q)])
        pltpu.sync_copy(id0, ci_sh.at[pl.ds(lr * WPR * Q, WPR * Q)])
        pltpu.sync_copy(ck_sh.at[pl.ds(lr * WPR * Q, WPR * Q)], kk1)
        pltpu.sync_copy(ci_sh.at[pl.ds(lr * WPR * Q, WPR * Q)], id1)
      return 0
    lax.fori_loop(0, 2, dense_pass, 0)

    def unkey(i, _):
      sl = pl.ds(i * 16, 16)
      kk = plsc.bitcast(kk0[sl], jnp.uint32)
      negk = kk >= SIGN
      u = jnp.where(negk, kk, (~kk) & NOSIGN)
      wtop[sl] = plsc.bitcast(u, jnp.float32)
      return 0
    lax.fori_loop(0, K // 16, unkey, 0, unroll=4)
    pltpu.sync_copy(wtop.at[pl.ds(0, K)], wout_hbm.at[pl.ds(r * K, K)])
    pltpu.sync_copy(id0.at[pl.ds(0, K)], idx_sh.at[pl.ds(lr * K, K)])
    pltpu.sync_copy(wtop.at[pl.ds(0, K)], w_sh.at[pl.ds(lr * K, K)])

  plsc.subcore_barrier()

  out_base = r * K + q * IPW
  pltpu.sync_copy(idx_sh.at[pl.ds(lr * K + q * IPW, IPW)], idx_v)
  pltpu.sync_copy(w_sh.at[pl.ds(lr * K + q * IPW, IPW)], wv)

  def glob(i, _):
    sl = pl.ds(i * 16, 16)
    gidx_v[sl] = idx_v[sl] + r * N
    return 0
  lax.fori_loop(0, IPW // 16, glob, 0, unroll=4)

  nch = IPW // CHUNK
  npairs = nch // 2
  FB = CHUNK * D * 4

  def _gissue(t, buf, gsem):
    return pltpu.async_copy(f_hbm.at[gidx_v.at[pl.ds(t * CHUNK, CHUNK)]],
                            buf, gsem)

  def _scale(buf, t):
    def rowscale(j, _):
      wj = plsc.load_gather(wv, [t * CHUNK + j + jnp.zeros((16,), jnp.int32)])
      def colscale(m, _):
        sl = pl.ds(m * 16, 16)
        buf[j, sl] = buf[j, sl] * wj
        return 0
      lax.fori_loop(0, D // 16, colscale, 0, unroll=8)
      return 0
    with jax.named_scope("p2_scale"):
      lax.fori_loop(0, CHUNK, rowscale, 0)

  def _wait(ref_from, ref_to, sm):
    pltpu.make_async_copy(ref_from, ref_to, sm).wait()

  _gissue(0, fbuf, sem)
  _gissue(1, fbuf2, gsem2)

  for cp in ccps:
    cp.wait()

  def cgather(i, _):
    j16 = iota + i * 16
    idx16 = idx_v[pl.ds(i * 16, 16)]
    for k3 in range(3):
      vals = plsc.load_gather(crow, [idx16 + (k3 * N)])
      plsc.store_scatter(cbuf, [j16 * 3 + k3], vals)
    return 0
  with jax.named_scope("p2_coords"):
    lax.fori_loop(0, IPW // 16, cgather, 0, unroll=2)
  pltpu.sync_copy(cbuf, cout_hbm.at[pl.ds(out_base * 3, IPW * 3)])

  def pairbody(t, _):
    o0 = out_base + (2 * t) * CHUNK
    o1 = out_base + (2 * t + 1) * CHUNK
    _wait(f_hbm.at[pl.ds(0, CHUNK)], fbuf, sem)
    _scale(fbuf, 2 * t)
    pltpu.async_copy(fbuf, fout_hbm.at[pl.ds(o0, CHUNK)], wsem)
    _wait(f_hbm.at[pl.ds(0, CHUNK)], fbuf2, gsem2)
    _scale(fbuf2, 2 * t + 1)
    pltpu.async_copy(fbuf2, fout_hbm.at[pl.ds(o1, CHUNK)], wsem2)

    @pl.when(t + 1 < npairs)
    def _():
      _wait(fbuf, fout_hbm.at[pl.ds(0, CHUNK)], wsem)
      _gissue(2 * t + 2, fbuf, sem)
      _wait(fbuf2, fout_hbm.at[pl.ds(0, CHUNK)], wsem2)
      _gissue(2 * t + 3, fbuf2, gsem2)
    return 0

  lax.fori_loop(0, npairs, pairbody, 0)
  _wait(fbuf, fout_hbm.at[pl.ds(0, CHUNK)], wsem)
  _wait(fbuf2, fout_hbm.at[pl.ds(0, CHUNK)], wsem2)


def _get_kernel():
  mesh = plsc.VectorSubcoreMesh(core_axis_name="c", subcore_axis_name="s",
                                num_cores=NC, num_subcores=NS)
  return pl.kernel(
      _sc_body,
      out_type=(jax.ShapeDtypeStruct((B * K, D), jnp.float32),
                jax.ShapeDtypeStruct((B * K * 3,), jnp.float32),
                jax.ShapeDtypeStruct((B * K,), jnp.float32)),
      mesh=mesh,
      compiler_params=pltpu.CompilerParams(needs_layout_passes=False),
      scratch_types=[
          pltpu.VMEM((N,), jnp.int32),
          pltpu.VMEM((N,), jnp.int32),
          pltpu.VMEM((N,), jnp.int32),
          pltpu.VMEM((N,), jnp.int32),
          pltpu.VMEM((XB,), jnp.int32),
          pltpu.VMEM((XB,), jnp.int32),
          pltpu.VMEM((SXB,), jnp.int32),
          pltpu.VMEM((SXB,), jnp.int32),
          pltpu.VMEM((Q,), jnp.float32),
          pltpu.VMEM((WPR * RADIX,), jnp.int32),
          pltpu.VMEM((WPR * 16,), jnp.int32),
          pltpu.VMEM((16,), jnp.int32),
          pltpu.VMEM((3 * N,), jnp.float32),
          pltpu.VMEM((IPW * 3,), jnp.float32),
          pltpu.VMEM((IPW,), jnp.int32),
          pltpu.VMEM((IPW,), jnp.int32),
          pltpu.VMEM((IPW,), jnp.float32),
          pltpu.VMEM((CHUNK, D), jnp.float32),
          pltpu.VMEM((CHUNK, D), jnp.float32),
          pltpu.VMEM_SHARED((NS * RADIX,), jnp.int32),
          pltpu.VMEM_SHARED((NS * 16,), jnp.int32),
          pltpu.VMEM_SHARED((NS * Q,), jnp.int32),
          pltpu.VMEM_SHARED((NS * Q,), jnp.int32),
          pltpu.VMEM_SHARED((RPC * K,), jnp.int32),
          pltpu.VMEM_SHARED((RPC * K,), jnp.float32),
          pltpu.SemaphoreType.DMA,
          pltpu.SemaphoreType.DMA,
          pltpu.SemaphoreType.DMA,
          pltpu.SemaphoreType.DMA,
          pltpu.SemaphoreType.DMA,
      ],
  )


def kernel(node_weights, node_feats, coords):
  w1 = node_weights.reshape(B * N)
  f2 = node_feats.reshape(B * N, D)
  ct = jnp.transpose(coords, (2, 0, 1)).reshape(3 * B * N)
  fout, cout, wout = _get_kernel()(w1, f2, ct)
  return (fout.reshape(B, K, D), cout.reshape(B, K, 3),
          wout.reshape(B, K))
